# baseline (speedup 1.0000x reference)
import jax
import jax.numpy as jnp
from jax.experimental import pallas as pl
from jax.experimental.pallas import tpu as pltpu

_TT = 32


def _joint_body(tn_ref, pn_ref, out_ref, pn_scratch):
    t = pl.program_id(1)

    @pl.when(t == 0)
    def _():
        pn = pn_ref[0]
        pn_max = jnp.max(pn, axis=-1, keepdims=True)
        pn_ls = pn - pn_max - jnp.log(
            jnp.sum(jnp.exp(pn - pn_max), axis=-1, keepdims=True))
        col = jax.lax.broadcasted_iota(jnp.int32, pn_ls.shape, 1)
        pn_scratch[...] = jnp.where(col == 0, 0.0, pn_ls)

    tn = tn_ref[0]
    tn_max = jnp.max(tn, axis=-1, keepdims=True)
    tn_ls = tn - tn_max - jnp.log(
        jnp.sum(jnp.exp(tn - tn_max), axis=-1, keepdims=True))

    out_ref[0] = pn_scratch[...][None, :, :] + tn_ls[:, None, :]


def kernel(tn_out, pn_out):
    B, T, V = tn_out.shape
    _, U, _ = pn_out.shape
    grid = (B, T // _TT)
    return pl.pallas_call(
        _joint_body,
        grid=grid,
        in_specs=[
            pl.BlockSpec((1, _TT, V), lambda b, t: (b, t, 0)),
            pl.BlockSpec((1, U, V), lambda b, t: (b, 0, 0)),
        ],
        out_specs=pl.BlockSpec((1, _TT, U, V), lambda b, t: (b, t, 0, 0)),
        out_shape=jax.ShapeDtypeStruct((B, T, U, V), tn_out.dtype),
        scratch_shapes=[pltpu.VMEM((U, V), tn_out.dtype)],
    )(tn_out, pn_out)


# speedup vs baseline: 1.1911x; 1.1911x over previous
import jax
import jax.numpy as jnp
from jax.experimental import pallas as pl
from jax.experimental.pallas import tpu as pltpu

_TT = 128


def _joint_body(tn_ref, pn_ref, out_ref, pn_scratch):
    t = pl.program_id(1)

    @pl.when(t == 0)
    def _():
        pn = pn_ref[0]
        pn_max = jnp.max(pn, axis=-1, keepdims=True)
        pn_ls = pn - pn_max - jnp.log(
            jnp.sum(jnp.exp(pn - pn_max), axis=-1, keepdims=True))
        col = jax.lax.broadcasted_iota(jnp.int32, pn_ls.shape, 1)
        pn_scratch[...] = jnp.where(col == 0, 0.0, pn_ls)

    tn = tn_ref[0]
    tn_max = jnp.max(tn, axis=-1, keepdims=True)
    tn_ls = tn - tn_max - jnp.log(
        jnp.sum(jnp.exp(tn - tn_max), axis=-1, keepdims=True))

    out_ref[0] = pn_scratch[...][None, :, :] + tn_ls[:, None, :]


def kernel(tn_out, pn_out):
    B, T, V = tn_out.shape
    _, U, _ = pn_out.shape
    grid = (B, T // _TT)
    return pl.pallas_call(
        _joint_body,
        grid=grid,
        in_specs=[
            pl.BlockSpec((1, _TT, V), lambda b, t: (b, t, 0)),
            pl.BlockSpec((1, U, V), lambda b, t: (b, 0, 0)),
        ],
        out_specs=pl.BlockSpec((1, _TT, U, V), lambda b, t: (b, t, 0, 0)),
        out_shape=jax.ShapeDtypeStruct((B, T, U, V), tn_out.dtype),
        scratch_shapes=[pltpu.VMEM((U, V), tn_out.dtype)],
    )(tn_out, pn_out)


# speedup vs baseline: 1.1952x; 1.0035x over previous
import jax
import jax.numpy as jnp
from jax.experimental import pallas as pl
from jax.experimental.pallas import tpu as pltpu

_K = 64
_N = 4


def _joint_body(tn_ref, pn_ref, out_ref, slots, pn_s, sems):
    b = pl.program_id(0)
    t = pl.program_id(1)
    nT = pl.num_programs(1)
    total = pl.num_programs(0) * nT
    s = b * nT + t
    slot = jax.lax.rem(s, _N)

    @pl.when(t == 0)
    def _():
        pn = pn_ref[0]
        pn_max = jnp.max(pn, axis=-1, keepdims=True)
        pn_ls = pn - pn_max - jnp.log(
            jnp.sum(jnp.exp(pn - pn_max), axis=-1, keepdims=True))
        col = jax.lax.broadcasted_iota(jnp.int32, pn_ls.shape, 1)
        pn_s[...] = jnp.where(col == 0, 0.0, pn_ls)

    dst = out_ref.at[b, pl.ds(t * _K, _K)]

    @pl.when(s >= _N)
    def _():
        pltpu.make_async_copy(slots.at[slot], dst, sems.at[slot]).wait()

    tn = tn_ref[0]
    tn_max = jnp.max(tn, axis=-1, keepdims=True)
    tn_ls = tn - tn_max - jnp.log(
        jnp.sum(jnp.exp(tn - tn_max), axis=-1, keepdims=True))

    slots[slot] = pn_s[...][None, :, :] + tn_ls[:, None, :]
    pltpu.make_async_copy(slots.at[slot], dst, sems.at[slot]).start()

    @pl.when(s == total - 1)
    def _():
        for i in range(_N):
            pltpu.make_async_copy(slots.at[i], dst, sems.at[i]).wait()


def kernel(tn_out, pn_out):
    B, T, V = tn_out.shape
    _, U, _ = pn_out.shape
    grid = (B, T // _K)
    return pl.pallas_call(
        _joint_body,
        grid=grid,
        in_specs=[
            pl.BlockSpec((1, _K, V), lambda b, t: (b, t, 0)),
            pl.BlockSpec((1, U, V), lambda b, t: (b, 0, 0)),
        ],
        out_specs=pl.BlockSpec(memory_space=pl.ANY),
        out_shape=jax.ShapeDtypeStruct((B, T, U, V), tn_out.dtype),
        scratch_shapes=[
            pltpu.VMEM((_N, _K, U, V), tn_out.dtype),
            pltpu.VMEM((U, V), tn_out.dtype),
            pltpu.SemaphoreType.DMA((_N,)),
        ],
        compiler_params=pltpu.CompilerParams(
            vmem_limit_bytes=100 * 1024 * 1024),
    )(tn_out, pn_out)


# speedup vs baseline: 1.1968x; 1.0014x over previous
import jax
import jax.numpy as jnp
from jax.experimental import pallas as pl
from jax.experimental.pallas import tpu as pltpu

_K = 64
_N = 8


def _joint_body(tn_ref, pn_ref, out_ref, slots, pn_s, sems):
    b = pl.program_id(0)
    t = pl.program_id(1)
    nT = pl.num_programs(1)
    total = pl.num_programs(0) * nT
    s = b * nT + t
    slot = jax.lax.rem(s, _N)

    @pl.when(t == 0)
    def _():
        pn = pn_ref[0]
        pn_max = jnp.max(pn, axis=-1, keepdims=True)
        pn_ls = pn - pn_max - jnp.log(
            jnp.sum(jnp.exp(pn - pn_max), axis=-1, keepdims=True))
        col = jax.lax.broadcasted_iota(jnp.int32, pn_ls.shape, 1)
        pn_s[...] = jnp.where(col == 0, 0.0, pn_ls)

    dst = out_ref.at[b, pl.ds(t * _K, _K)]

    @pl.when(s >= _N)
    def _():
        pltpu.make_async_copy(slots.at[slot], dst, sems.at[slot]).wait()

    tn = tn_ref[0]
    tn_max = jnp.max(tn, axis=-1, keepdims=True)
    tn_ls = tn - tn_max - jnp.log(
        jnp.sum(jnp.exp(tn - tn_max), axis=-1, keepdims=True))

    slots[slot] = pn_s[...][None, :, :] + tn_ls[:, None, :]
    for i in range(_N):
        @pl.when(slot == i)
        def _(i=i):
            pltpu.make_async_copy(slots.at[i], dst, sems.at[i]).start()

    @pl.when(s == total - 1)
    def _():
        for i in range(_N):
            pltpu.make_async_copy(slots.at[i], dst, sems.at[i]).wait()


def kernel(tn_out, pn_out):
    B, T, V = tn_out.shape
    _, U, _ = pn_out.shape
    grid = (B, T // _K)
    return pl.pallas_call(
        _joint_body,
        grid=grid,
        in_specs=[
            pl.BlockSpec((1, _K, V), lambda b, t: (b, t, 0)),
            pl.BlockSpec((1, U, V), lambda b, t: (b, 0, 0)),
        ],
        out_specs=pl.BlockSpec(memory_space=pl.ANY),
        out_shape=jax.ShapeDtypeStruct((B, T, U, V), tn_out.dtype),
        scratch_shapes=[
            pltpu.VMEM((_N, _K, U, V), tn_out.dtype),
            pltpu.VMEM((U, V), tn_out.dtype),
            pltpu.SemaphoreType.DMA((_N,)),
        ],
        compiler_params=pltpu.CompilerParams(
            vmem_limit_bytes=100 * 1024 * 1024),
    )(tn_out, pn_out)
